# baseline (speedup 1.0000x reference)
import jax
import jax.numpy as jnp
from jax.experimental import pallas as pl


_BLOCK_ROWS = 8000


def _broadcast_body(w_ref, o_ref):
    o_ref[...] = jnp.broadcast_to(w_ref[...], o_ref.shape)


def kernel(edge_index, W):
    E = edge_index.shape[1]
    grid = (E // _BLOCK_ROWS,)
    return pl.pallas_call(
        _broadcast_body,
        grid=grid,
        in_specs=[pl.BlockSpec((1, 64), lambda i: (0, 0))],
        out_specs=pl.BlockSpec((_BLOCK_ROWS, 64), lambda i: (i, 0)),
        out_shape=jax.ShapeDtypeStruct((E, 64), jnp.float32),
    )(W)
